# baseline (speedup 1.0000x reference)
import functools

import jax
import jax.numpy as jnp
from jax import lax
from jax.experimental import pallas as pl
from jax.experimental.pallas import tpu as pltpu
from jax.experimental.pallas import tpu_sc as plsc

_NC = 2
_NS = 16
_NW = _NC * _NS
_LANES = 16
_CW = 128


def _pe_table(seq_len, dim):
    pos = jnp.arange(seq_len, dtype=jnp.float32)[:, None]
    i = jnp.arange(0, dim, 2, dtype=jnp.float32)
    div = jnp.exp(-(jnp.log(10000.0)) * i / dim)
    pe = jnp.zeros((seq_len, dim), dtype=jnp.float32)
    pe = pe.at[:, 0::2].set(jnp.sin(pos * div))
    pe = pe.at[:, 1::2].set(jnp.cos(pos * div))
    return pe


@jax.jit
def _relayout(tt, tail3):
    D, V = tt.shape
    NF = V // _CW
    BASE = NF // _NW
    EXTRA = NF % _NW
    W = 4
    NG = BASE // W
    CPG = W * _CW
    RPG = CPG // 2

    mesh = plsc.VectorSubcoreMesh(core_axis_name="c", subcore_axis_name="s")

    @functools.partial(
        pl.kernel,
        mesh=mesh,
        compiler_params=pltpu.CompilerParams(
            use_tc_tiling_on_sc=True, needs_layout_passes=False
        ),
        out_type=jax.ShapeDtypeStruct((V * D // 128, 128), jnp.float32),
        scratch_types=[
            [pltpu.VMEM((D, CPG), jnp.float32) for _ in range(2)],
            [pltpu.VMEM((RPG // 2, 128), jnp.float32) for _ in range(2)],
            [pltpu.SemaphoreType.DMA for _ in range(2)],
            [pltpu.SemaphoreType.DMA for _ in range(2)],
        ],
    )
    def ka(tt_hbm, tail_hbm, out_hbm, wbufs, rbufs, gsem, ssem):
        wid = lax.axis_index("s") * _NC + lax.axis_index("c")
        start = wid * BASE + jnp.minimum(wid, EXTRA)
        d_iota = lax.iota(jnp.int32, _LANES)
        rows_k = [d_iota + k * _LANES for k in range(D // _LANES)]

        def fire_group(g, b, ncol=CPG):
            v0 = (start + g * W) * _CW
            for gg in range(D // 8):
                pltpu.async_copy(
                    tt_hbm.at[pl.ds(8 * gg, 8), pl.ds(v0, ncol)],
                    wbufs[b].at[pl.ds(8 * gg, 8), pl.ds(0, ncol)],
                    gsem[b],
                )

        def wait_group(b, ncol=CPG):
            for gg in range(D // 8):
                pltpu.make_async_copy(
                    tt_hbm.at[pl.ds(0, 8), pl.ds(0, ncol)],
                    wbufs[b].at[pl.ds(0, 8), pl.ds(0, ncol)],
                    gsem[b],
                ).wait()

        def transpose_half(b, h):
            wb = wbufs[b]
            rb = rbufs[h]

            def tblk(t, carry):
                c0 = h * (CPG // 2) + t * 8
                cols = [jnp.full((_LANES,), c0 + i, jnp.int32) for i in range(8)]
                for i in range(8):
                    for k in range(D // _LANES):
                        vals = plsc.load_gather(wb, [rows_k[k], cols[i]])
                        rb[t * 4 + (i >> 1), pl.ds((i & 1) * D + k * _LANES, _LANES)] = vals
                return carry

            lax.fori_loop(0, CPG // 16, tblk, 0)

        def fire_store(g, h):
            r0 = (start + g * W) * (_CW // 2) + h * (RPG // 2)
            return pltpu.async_copy(
                rbufs[h], out_hbm.at[pl.ds(r0, RPG // 2)], ssem[h]
            )

        def wait_store(h):
            pltpu.make_async_copy(
                rbufs[h], out_hbm.at[pl.ds(0, RPG // 2)], ssem[h]
            ).wait()

        fire_group(0, 0)
        fire_group(1, 1)

        def pairs(gp, carry):
            for b in range(2):
                g = 2 * gp + b
                wait_group(b)
                for h in range(2):
                    @pl.when(g > 0)
                    def _():
                        wait_store(h)

                    transpose_half(b, h)
                    fire_store(g, h)

                @pl.when(g + 2 < NG)
                def _():
                    fire_group(g + 2, b)

            return carry

        lax.fori_loop(0, NG // 2, pairs, 0)

        if NG % 2 == 1:
            g_last = NG - 1
            wait_group(0)
            for h in range(2):
                wait_store(h)
                transpose_half(0, h)
                fire_store(g_last, h)
        wait_store(0)
        wait_store(1)

        @pl.when(wid < EXTRA)
        def _():
            v0 = (start + BASE) * _CW
            for gg in range(D // 8):
                pltpu.async_copy(
                    tt_hbm.at[pl.ds(8 * gg, 8), pl.ds(v0, _CW)],
                    wbufs[0].at[pl.ds(8 * gg, 8), pl.ds(0, _CW)],
                    gsem[0],
                )
            wait_group(0, ncol=_CW)

            def xblk(t, carry):
                c0 = t * 8
                cols = [jnp.full((_LANES,), c0 + i, jnp.int32) for i in range(8)]
                for i in range(8):
                    for k in range(D // _LANES):
                        vals = plsc.load_gather(wbufs[0], [rows_k[k], cols[i]])
                        rbufs[0][t * 4 + (i >> 1), pl.ds((i & 1) * D + k * _LANES, _LANES)] = vals
                return carry

            lax.fori_loop(0, _CW // 8, xblk, 0)
            pltpu.sync_copy(
                rbufs[0].at[pl.ds(0, _CW // 2), :],
                out_hbm.at[pl.ds((start + BASE) * (_CW // 2), _CW // 2)],
            )

        @pl.when(wid == EXTRA)
        def _():
            t = tail_hbm.shape[0]
            pltpu.sync_copy(tail_hbm, rbufs[1].at[pl.ds(0, t), :])
            pltpu.sync_copy(
                rbufs[1].at[pl.ds(0, t), :],
                out_hbm.at[pl.ds(NF * (_CW // 2), t)],
            )

    return ka(tt, tail3)


@functools.partial(jax.jit, static_argnums=(3, 4))
def _emb_lookup(idx, pe4, table, seq_len, nch):
    D = table.shape[1]
    mesh = plsc.VectorSubcoreMesh(core_axis_name="c", subcore_axis_name="s")

    NBUF = 10
    H = 5

    @functools.partial(
        pl.kernel,
        mesh=mesh,
        compiler_params=pltpu.CompilerParams(use_tc_tiling_on_sc=False),
        out_type=jax.ShapeDtypeStruct((_NW * nch * _CW, D), jnp.float32),
        scratch_types=[
            pltpu.VMEM((nch, _CW), jnp.int32),
            pltpu.VMEM((4 * seq_len, D), jnp.float32),
            [pltpu.VMEM((_CW, D), jnp.float32) for _ in range(NBUF)],
            [pltpu.SemaphoreType.DMA for _ in range(NBUF)],
            [pltpu.SemaphoreType.DMA for _ in range(NBUF)],
        ],
    )
    def k(idx_hbm, pe4_hbm, table_hbm, out_hbm, idx_v, pe4_v, bufs, gsem, ssem):
        wid = lax.axis_index("s") * _NC + lax.axis_index("c")
        pltpu.sync_copy(idx_hbm.at[wid], idx_v)
        pltpu.sync_copy(pe4_hbm, pe4_v)

        def fire_gather(j):
            b = j % NBUF
            return pltpu.async_copy(table_hbm.at[idx_v.at[j]], bufs[b], gsem[b])

        def add_pe(j):
            buf = bufs[j % NBUF]
            phase = (j * _CW) % seq_len

            def row(r, c2):
                for c in range(D // _LANES):
                    sl = pl.ds(c * _LANES, _LANES)
                    buf[r, sl] = buf[r, sl] + pe4_v[phase + r, sl]
                return c2

            lax.fori_loop(0, _CW, row, 0, unroll=2)

        gh = [None] * nch
        sh = [None] * nch
        for j in range(H):
            gh[j] = fire_gather(j)
        for j in range(nch):
            b = j % NBUF
            gh[j].wait()
            add_pe(j)
            sh[j] = pltpu.async_copy(
                bufs[b], out_hbm.at[pl.ds((wid * nch + j) * _CW, _CW)], ssem[b]
            )
            jn = j + H
            if jn < nch:
                if jn - NBUF >= 0:
                    sh[jn - NBUF].wait()
                gh[jn] = fire_gather(jn)
        for j in range(max(0, nch - NBUF), nch):
            sh[j].wait()

    return k(idx, pe4, table)


def kernel(sequences, token_table):
    B, L = sequences.shape
    V, D = token_table.shape
    R = B * L
    assert R % (_NW * _CW) == 0
    nch = R // (_NW * _CW)

    VF = (V // _CW) * _CW
    tail3 = token_table[VF:].reshape((V - VF) * D // 128, 128)
    tab_lin = _relayout(token_table.T, tail3)
    tab_lin = tab_lin.reshape(V, D)

    idx = sequences.reshape(_NW, nch, _CW).astype(jnp.int32)
    pe = _pe_table(L, D)
    pe4 = jnp.concatenate([pe, pe, pe, pe], axis=0)
    out = _emb_lookup(idx, pe4, tab_lin, L, nch)
    return out.reshape(B, L, D)


# speedup vs baseline: 1.0012x; 1.0012x over previous
import functools

import jax
import jax.numpy as jnp
from jax import lax
from jax.experimental import pallas as pl
from jax.experimental.pallas import tpu as pltpu
from jax.experimental.pallas import tpu_sc as plsc

_NC = 2
_NS = 16
_NW = _NC * _NS
_LANES = 16
_CW = 128


def _pe_table(seq_len, dim):
    pos = jnp.arange(seq_len, dtype=jnp.float32)[:, None]
    i = jnp.arange(0, dim, 2, dtype=jnp.float32)
    div = jnp.exp(-(jnp.log(10000.0)) * i / dim)
    pe = jnp.zeros((seq_len, dim), dtype=jnp.float32)
    pe = pe.at[:, 0::2].set(jnp.sin(pos * div))
    pe = pe.at[:, 1::2].set(jnp.cos(pos * div))
    return pe


@jax.jit
def _relayout(tt, tail3):
    D, V = tt.shape
    NF = V // _CW
    BASE = NF // _NW
    EXTRA = NF % _NW
    W = 4
    NG = BASE // W
    CPG = W * _CW
    RPG = CPG // 2

    mesh = plsc.VectorSubcoreMesh(core_axis_name="c", subcore_axis_name="s")

    @functools.partial(
        pl.kernel,
        mesh=mesh,
        compiler_params=pltpu.CompilerParams(
            use_tc_tiling_on_sc=True, needs_layout_passes=False
        ),
        out_type=jax.ShapeDtypeStruct((V * D // 128, 128), jnp.float32),
        scratch_types=[
            [pltpu.VMEM((D, CPG + 1), jnp.float32) for _ in range(2)],
            [pltpu.VMEM((RPG // 2, 128), jnp.float32) for _ in range(2)],
            [pltpu.SemaphoreType.DMA for _ in range(2)],
            [pltpu.SemaphoreType.DMA for _ in range(2)],
        ],
    )
    def ka(tt_hbm, tail_hbm, out_hbm, wbufs, rbufs, gsem, ssem):
        wid = lax.axis_index("s") * _NC + lax.axis_index("c")
        start = wid * BASE + jnp.minimum(wid, EXTRA)
        d_iota = lax.iota(jnp.int32, _LANES)
        rows_k = [d_iota + k * _LANES for k in range(D // _LANES)]

        def fire_group(g, b, ncol=CPG):
            v0 = (start + g * W) * _CW
            for gg in range(D // 8):
                pltpu.async_copy(
                    tt_hbm.at[pl.ds(8 * gg, 8), pl.ds(v0, ncol)],
                    wbufs[b].at[pl.ds(8 * gg, 8), pl.ds(0, ncol)],
                    gsem[b],
                )

        def wait_group(b, ncol=CPG):
            for gg in range(D // 8):
                pltpu.make_async_copy(
                    tt_hbm.at[pl.ds(0, 8), pl.ds(0, ncol)],
                    wbufs[b].at[pl.ds(0, 8), pl.ds(0, ncol)],
                    gsem[b],
                ).wait()

        def transpose_half(b, h):
            wb = wbufs[b]
            rb = rbufs[h]

            def tblk(t, carry):
                c0 = h * (CPG // 2) + t * 8
                cols = [jnp.full((_LANES,), c0 + i, jnp.int32) for i in range(8)]
                for i in range(8):
                    for k in range(D // _LANES):
                        vals = plsc.load_gather(wb, [rows_k[k], cols[i]])
                        rb[t * 4 + (i >> 1), pl.ds((i & 1) * D + k * _LANES, _LANES)] = vals
                return carry

            lax.fori_loop(0, CPG // 16, tblk, 0)

        def fire_store(g, h):
            r0 = (start + g * W) * (_CW // 2) + h * (RPG // 2)
            return pltpu.async_copy(
                rbufs[h], out_hbm.at[pl.ds(r0, RPG // 2)], ssem[h]
            )

        def wait_store(h):
            pltpu.make_async_copy(
                rbufs[h], out_hbm.at[pl.ds(0, RPG // 2)], ssem[h]
            ).wait()

        fire_group(0, 0)
        fire_group(1, 1)

        def pairs(gp, carry):
            for b in range(2):
                g = 2 * gp + b
                wait_group(b)
                for h in range(2):
                    @pl.when(g > 0)
                    def _():
                        wait_store(h)

                    transpose_half(b, h)
                    fire_store(g, h)

                @pl.when(g + 2 < NG)
                def _():
                    fire_group(g + 2, b)

            return carry

        lax.fori_loop(0, NG // 2, pairs, 0)

        if NG % 2 == 1:
            g_last = NG - 1
            wait_group(0)
            for h in range(2):
                wait_store(h)
                transpose_half(0, h)
                fire_store(g_last, h)
        wait_store(0)
        wait_store(1)

        @pl.when(wid < EXTRA)
        def _():
            v0 = (start + BASE) * _CW
            for gg in range(D // 8):
                pltpu.async_copy(
                    tt_hbm.at[pl.ds(8 * gg, 8), pl.ds(v0, _CW)],
                    wbufs[0].at[pl.ds(8 * gg, 8), pl.ds(0, _CW)],
                    gsem[0],
                )
            wait_group(0, ncol=_CW)

            def xblk(t, carry):
                c0 = t * 8
                cols = [jnp.full((_LANES,), c0 + i, jnp.int32) for i in range(8)]
                for i in range(8):
                    for k in range(D // _LANES):
                        vals = plsc.load_gather(wbufs[0], [rows_k[k], cols[i]])
                        rbufs[0][t * 4 + (i >> 1), pl.ds((i & 1) * D + k * _LANES, _LANES)] = vals
                return carry

            lax.fori_loop(0, _CW // 8, xblk, 0)
            pltpu.sync_copy(
                rbufs[0].at[pl.ds(0, _CW // 2), :],
                out_hbm.at[pl.ds((start + BASE) * (_CW // 2), _CW // 2)],
            )

        @pl.when(wid == EXTRA)
        def _():
            t = tail_hbm.shape[0]
            pltpu.sync_copy(tail_hbm, rbufs[1].at[pl.ds(0, t), :])
            pltpu.sync_copy(
                rbufs[1].at[pl.ds(0, t), :],
                out_hbm.at[pl.ds(NF * (_CW // 2), t)],
            )

    return ka(tt, tail3)


@functools.partial(jax.jit, static_argnums=(3, 4))
def _emb_lookup(idx, pe4, table, seq_len, nch):
    D = table.shape[1]
    mesh = plsc.VectorSubcoreMesh(core_axis_name="c", subcore_axis_name="s")

    NBUF = 10
    H = 5

    @functools.partial(
        pl.kernel,
        mesh=mesh,
        compiler_params=pltpu.CompilerParams(use_tc_tiling_on_sc=False),
        out_type=jax.ShapeDtypeStruct((_NW * nch * _CW, D), jnp.float32),
        scratch_types=[
            pltpu.VMEM((nch, _CW), jnp.int32),
            pltpu.VMEM((4 * seq_len, D), jnp.float32),
            [pltpu.VMEM((_CW, D), jnp.float32) for _ in range(NBUF)],
            [pltpu.SemaphoreType.DMA for _ in range(NBUF)],
            [pltpu.SemaphoreType.DMA for _ in range(NBUF)],
        ],
    )
    def k(idx_hbm, pe4_hbm, table_hbm, out_hbm, idx_v, pe4_v, bufs, gsem, ssem):
        wid = lax.axis_index("s") * _NC + lax.axis_index("c")
        pltpu.sync_copy(idx_hbm.at[wid], idx_v)
        pltpu.sync_copy(pe4_hbm, pe4_v)

        def fire_gather(j):
            b = j % NBUF
            return pltpu.async_copy(table_hbm.at[idx_v.at[j]], bufs[b], gsem[b])

        def add_pe(j):
            buf = bufs[j % NBUF]
            phase = (j * _CW) % seq_len

            def row(r, c2):
                for c in range(D // _LANES):
                    sl = pl.ds(c * _LANES, _LANES)
                    buf[r, sl] = buf[r, sl] + pe4_v[phase + r, sl]
                return c2

            lax.fori_loop(0, _CW, row, 0, unroll=2)

        gh = [None] * nch
        sh = [None] * nch
        for j in range(H):
            gh[j] = fire_gather(j)
        for j in range(nch):
            b = j % NBUF
            gh[j].wait()
            add_pe(j)
            sh[j] = pltpu.async_copy(
                bufs[b], out_hbm.at[pl.ds((wid * nch + j) * _CW, _CW)], ssem[b]
            )
            jn = j + H
            if jn < nch:
                if jn - NBUF >= 0:
                    sh[jn - NBUF].wait()
                gh[jn] = fire_gather(jn)
        for j in range(max(0, nch - NBUF), nch):
            sh[j].wait()

    return k(idx, pe4, table)


def kernel(sequences, token_table):
    B, L = sequences.shape
    V, D = token_table.shape
    R = B * L
    assert R % (_NW * _CW) == 0
    nch = R // (_NW * _CW)

    VF = (V // _CW) * _CW
    tail3 = token_table[VF:].reshape((V - VF) * D // 128, 128)
    tab_lin = _relayout(token_table.T, tail3)
    tab_lin = tab_lin.reshape(V, D)

    idx = sequences.reshape(_NW, nch, _CW).astype(jnp.int32)
    pe = _pe_table(L, D)
    pe4 = jnp.concatenate([pe, pe, pe, pe], axis=0)
    out = _emb_lookup(idx, pe4, tab_lin, L, nch)
    return out.reshape(B, L, D)


# speedup vs baseline: 1.6374x; 1.6355x over previous
import functools

import jax
import jax.numpy as jnp
from jax import lax
from jax.experimental import pallas as pl
from jax.experimental.pallas import tpu as pltpu
from jax.experimental.pallas import tpu_sc as plsc

_NC = 2
_NS = 16
_NW = _NC * _NS
_LANES = 16
_CW = 128


def _pe_table(seq_len, dim):
    pos = jnp.arange(seq_len, dtype=jnp.float32)[:, None]
    i = jnp.arange(0, dim, 2, dtype=jnp.float32)
    div = jnp.exp(-(jnp.log(10000.0)) * i / dim)
    pe = jnp.zeros((seq_len, dim), dtype=jnp.float32)
    pe = pe.at[:, 0::2].set(jnp.sin(pos * div))
    pe = pe.at[:, 1::2].set(jnp.cos(pos * div))
    return pe


@jax.jit
def _relayout(tt, tail3):
    D, V = tt.shape
    NF = V // _CW
    BASE = NF // _NW
    EXTRA = NF % _NW
    W = 4
    NG = BASE // W
    CPG = W * _CW
    RPG = CPG // 2

    mesh = plsc.VectorSubcoreMesh(core_axis_name="c", subcore_axis_name="s")

    @functools.partial(
        pl.kernel,
        mesh=mesh,
        compiler_params=pltpu.CompilerParams(
            use_tc_tiling_on_sc=True, needs_layout_passes=False
        ),
        out_type=jax.ShapeDtypeStruct((V * D // 128, 128), jnp.float32),
        scratch_types=[
            [pltpu.VMEM((D, CPG + 1), jnp.float32) for _ in range(2)],
            [pltpu.VMEM((RPG // 2, 128), jnp.float32) for _ in range(2)],
            [pltpu.SemaphoreType.DMA for _ in range(2)],
            [pltpu.SemaphoreType.DMA for _ in range(2)],
        ],
    )
    def ka(tt_hbm, tail_hbm, out_hbm, wbufs, rbufs, gsem, ssem):
        wid = lax.axis_index("s") * _NC + lax.axis_index("c")
        start = wid * BASE + jnp.minimum(wid, EXTRA)
        d_iota = lax.iota(jnp.int32, _LANES)
        rows_k = [d_iota + k * _LANES for k in range(D // _LANES)]

        def fire_group(g, b, ncol=CPG):
            v0 = (start + g * W) * _CW
            for gg in range(D // 8):
                pltpu.async_copy(
                    tt_hbm.at[pl.ds(8 * gg, 8), pl.ds(v0, ncol)],
                    wbufs[b].at[pl.ds(8 * gg, 8), pl.ds(0, ncol)],
                    gsem[b],
                )

        def wait_group(b, ncol=CPG):
            for gg in range(D // 8):
                pltpu.make_async_copy(
                    tt_hbm.at[pl.ds(0, 8), pl.ds(0, ncol)],
                    wbufs[b].at[pl.ds(0, 8), pl.ds(0, ncol)],
                    gsem[b],
                ).wait()

        def transpose_half(b, h):
            wb = wbufs[b]
            rb = rbufs[h]

            @plsc.parallel_loop(0, CPG // 16, unroll=2)
            def tblk(t):
                c0 = h * (CPG // 2) + t * 8
                cols = [jnp.full((_LANES,), c0 + i, jnp.int32) for i in range(8)]
                for i in range(8):
                    for k in range(D // _LANES):
                        vals = plsc.load_gather(wb, [rows_k[k], cols[i]])
                        rb[t * 4 + (i >> 1), pl.ds((i & 1) * D + k * _LANES, _LANES)] = vals

        def fire_store(g, h):
            r0 = (start + g * W) * (_CW // 2) + h * (RPG // 2)
            return pltpu.async_copy(
                rbufs[h], out_hbm.at[pl.ds(r0, RPG // 2)], ssem[h]
            )

        def wait_store(h):
            pltpu.make_async_copy(
                rbufs[h], out_hbm.at[pl.ds(0, RPG // 2)], ssem[h]
            ).wait()

        fire_group(0, 0)
        fire_group(1, 1)

        def pairs(gp, carry):
            for b in range(2):
                g = 2 * gp + b
                wait_group(b)
                for h in range(2):
                    @pl.when(g > 0)
                    def _():
                        wait_store(h)

                    transpose_half(b, h)
                    fire_store(g, h)

                @pl.when(g + 2 < NG)
                def _():
                    fire_group(g + 2, b)

            return carry

        lax.fori_loop(0, NG // 2, pairs, 0)

        if NG % 2 == 1:
            g_last = NG - 1
            wait_group(0)
            for h in range(2):
                wait_store(h)
                transpose_half(0, h)
                fire_store(g_last, h)
        wait_store(0)
        wait_store(1)

        @pl.when(wid < EXTRA)
        def _():
            v0 = (start + BASE) * _CW
            for gg in range(D // 8):
                pltpu.async_copy(
                    tt_hbm.at[pl.ds(8 * gg, 8), pl.ds(v0, _CW)],
                    wbufs[0].at[pl.ds(8 * gg, 8), pl.ds(0, _CW)],
                    gsem[0],
                )
            wait_group(0, ncol=_CW)

            @plsc.parallel_loop(0, _CW // 8, unroll=2)
            def xblk(t):
                c0 = t * 8
                cols = [jnp.full((_LANES,), c0 + i, jnp.int32) for i in range(8)]
                for i in range(8):
                    for k in range(D // _LANES):
                        vals = plsc.load_gather(wbufs[0], [rows_k[k], cols[i]])
                        rbufs[0][t * 4 + (i >> 1), pl.ds((i & 1) * D + k * _LANES, _LANES)] = vals
            pltpu.sync_copy(
                rbufs[0].at[pl.ds(0, _CW // 2), :],
                out_hbm.at[pl.ds((start + BASE) * (_CW // 2), _CW // 2)],
            )

        @pl.when(wid == EXTRA)
        def _():
            t = tail_hbm.shape[0]
            pltpu.sync_copy(tail_hbm, rbufs[1].at[pl.ds(0, t), :])
            pltpu.sync_copy(
                rbufs[1].at[pl.ds(0, t), :],
                out_hbm.at[pl.ds(NF * (_CW // 2), t)],
            )

    return ka(tt, tail3)


@functools.partial(jax.jit, static_argnums=(3, 4))
def _emb_lookup(idx, pe4, table, seq_len, nch):
    D = table.shape[1]
    mesh = plsc.VectorSubcoreMesh(core_axis_name="c", subcore_axis_name="s")

    NBUF = 10
    H = 5

    @functools.partial(
        pl.kernel,
        mesh=mesh,
        compiler_params=pltpu.CompilerParams(use_tc_tiling_on_sc=False),
        out_type=jax.ShapeDtypeStruct((_NW * nch * _CW, D), jnp.float32),
        scratch_types=[
            pltpu.VMEM((nch, _CW), jnp.int32),
            pltpu.VMEM((4 * seq_len, D), jnp.float32),
            [pltpu.VMEM((_CW, D), jnp.float32) for _ in range(NBUF)],
            [pltpu.SemaphoreType.DMA for _ in range(NBUF)],
            [pltpu.SemaphoreType.DMA for _ in range(NBUF)],
        ],
    )
    def k(idx_hbm, pe4_hbm, table_hbm, out_hbm, idx_v, pe4_v, bufs, gsem, ssem):
        wid = lax.axis_index("s") * _NC + lax.axis_index("c")
        pltpu.sync_copy(idx_hbm.at[wid], idx_v)
        pltpu.sync_copy(pe4_hbm, pe4_v)

        def fire_gather(j):
            b = j % NBUF
            return pltpu.async_copy(table_hbm.at[idx_v.at[j]], bufs[b], gsem[b])

        def add_pe(j):
            buf = bufs[j % NBUF]
            phase = (j * _CW) % seq_len

            def row(r, c2):
                for c in range(D // _LANES):
                    sl = pl.ds(c * _LANES, _LANES)
                    buf[r, sl] = buf[r, sl] + pe4_v[phase + r, sl]
                return c2

            lax.fori_loop(0, _CW, row, 0, unroll=2)

        gh = [None] * nch
        sh = [None] * nch
        for j in range(H):
            gh[j] = fire_gather(j)
        for j in range(nch):
            b = j % NBUF
            gh[j].wait()
            add_pe(j)
            sh[j] = pltpu.async_copy(
                bufs[b], out_hbm.at[pl.ds((wid * nch + j) * _CW, _CW)], ssem[b]
            )
            jn = j + H
            if jn < nch:
                if jn - NBUF >= 0:
                    sh[jn - NBUF].wait()
                gh[jn] = fire_gather(jn)
        for j in range(max(0, nch - NBUF), nch):
            sh[j].wait()

    return k(idx, pe4, table)


def kernel(sequences, token_table):
    B, L = sequences.shape
    V, D = token_table.shape
    R = B * L
    assert R % (_NW * _CW) == 0
    nch = R // (_NW * _CW)

    VF = (V // _CW) * _CW
    tail3 = token_table[VF:].reshape((V - VF) * D // 128, 128)
    tab_lin = _relayout(token_table.T, tail3)
    tab_lin = tab_lin.reshape(V, D)

    idx = sequences.reshape(_NW, nch, _CW).astype(jnp.int32)
    pe = _pe_table(L, D)
    pe4 = jnp.concatenate([pe, pe, pe, pe], axis=0)
    out = _emb_lookup(idx, pe4, tab_lin, L, nch)
    return out.reshape(B, L, D)


# speedup vs baseline: 1.7043x; 1.0409x over previous
import functools

import jax
import jax.numpy as jnp
from jax import lax
from jax.experimental import pallas as pl
from jax.experimental.pallas import tpu as pltpu
from jax.experimental.pallas import tpu_sc as plsc

_NC = 2
_NS = 16
_NW = _NC * _NS
_LANES = 16
_CW = 128


def _pe_table(seq_len, dim):
    pos = jnp.arange(seq_len, dtype=jnp.float32)[:, None]
    i = jnp.arange(0, dim, 2, dtype=jnp.float32)
    div = jnp.exp(-(jnp.log(10000.0)) * i / dim)
    pe = jnp.zeros((seq_len, dim), dtype=jnp.float32)
    pe = pe.at[:, 0::2].set(jnp.sin(pos * div))
    pe = pe.at[:, 1::2].set(jnp.cos(pos * div))
    return pe


@jax.jit
def _relayout(tt, tail3):
    D, V = tt.shape
    NF = V // _CW
    BASE = NF // _NW
    EXTRA = NF % _NW
    W = 4
    NG = BASE // W
    CPG = W * _CW
    RPG = CPG // 2

    mesh = plsc.VectorSubcoreMesh(core_axis_name="c", subcore_axis_name="s")

    @functools.partial(
        pl.kernel,
        mesh=mesh,
        compiler_params=pltpu.CompilerParams(
            use_tc_tiling_on_sc=True, needs_layout_passes=False
        ),
        out_type=jax.ShapeDtypeStruct((V * D // 128, 128), jnp.float32),
        scratch_types=[
            [pltpu.VMEM((D, CPG + 1), jnp.float32) for _ in range(2)],
            [pltpu.VMEM((RPG // 2, 128), jnp.float32) for _ in range(2)],
            [pltpu.SemaphoreType.DMA for _ in range(2)],
            [pltpu.SemaphoreType.DMA for _ in range(2)],
        ],
    )
    def ka(tt_hbm, tail_hbm, out_hbm, wbufs, rbufs, gsem, ssem):
        wid = lax.axis_index("s") * _NC + lax.axis_index("c")
        start = wid * BASE + jnp.minimum(wid, EXTRA)
        d_iota = lax.iota(jnp.int32, _LANES)
        rows_k = [d_iota + k * _LANES for k in range(D // _LANES)]

        def fire_group(g, b, ncol=CPG):
            v0 = (start + g * W) * _CW
            for gg in range(D // 8):
                pltpu.async_copy(
                    tt_hbm.at[pl.ds(8 * gg, 8), pl.ds(v0, ncol)],
                    wbufs[b].at[pl.ds(8 * gg, 8), pl.ds(0, ncol)],
                    gsem[b],
                )

        def wait_group(b, ncol=CPG):
            for gg in range(D // 8):
                pltpu.make_async_copy(
                    tt_hbm.at[pl.ds(0, 8), pl.ds(0, ncol)],
                    wbufs[b].at[pl.ds(0, 8), pl.ds(0, ncol)],
                    gsem[b],
                ).wait()

        def transpose_half(b, h):
            wb = wbufs[b]
            rb = rbufs[h]

            @plsc.parallel_loop(0, CPG // 16, unroll=4)
            def tblk(t):
                c0 = h * (CPG // 2) + t * 8
                cols = [jnp.full((_LANES,), c0 + i, jnp.int32) for i in range(8)]
                for i in range(8):
                    for k in range(D // _LANES):
                        vals = plsc.load_gather(wb, [rows_k[k], cols[i]])
                        rb[t * 4 + (i >> 1), pl.ds((i & 1) * D + k * _LANES, _LANES)] = vals

        def fire_store(g, h):
            r0 = (start + g * W) * (_CW // 2) + h * (RPG // 2)
            return pltpu.async_copy(
                rbufs[h], out_hbm.at[pl.ds(r0, RPG // 2)], ssem[h]
            )

        def wait_store(h):
            pltpu.make_async_copy(
                rbufs[h], out_hbm.at[pl.ds(0, RPG // 2)], ssem[h]
            ).wait()

        fire_group(0, 0)
        fire_group(1, 1)

        def pairs(gp, carry):
            for b in range(2):
                g = 2 * gp + b
                wait_group(b)
                for h in range(2):
                    @pl.when(g > 0)
                    def _():
                        wait_store(h)

                    transpose_half(b, h)
                    fire_store(g, h)

                @pl.when(g + 2 < NG)
                def _():
                    fire_group(g + 2, b)

            return carry

        lax.fori_loop(0, NG // 2, pairs, 0)

        if NG % 2 == 1:
            g_last = NG - 1
            wait_group(0)
            for h in range(2):
                wait_store(h)
                transpose_half(0, h)
                fire_store(g_last, h)
        wait_store(0)
        wait_store(1)

        @pl.when(wid < EXTRA)
        def _():
            v0 = (start + BASE) * _CW
            for gg in range(D // 8):
                pltpu.async_copy(
                    tt_hbm.at[pl.ds(8 * gg, 8), pl.ds(v0, _CW)],
                    wbufs[0].at[pl.ds(8 * gg, 8), pl.ds(0, _CW)],
                    gsem[0],
                )
            wait_group(0, ncol=_CW)

            @plsc.parallel_loop(0, _CW // 8, unroll=4)
            def xblk(t):
                c0 = t * 8
                cols = [jnp.full((_LANES,), c0 + i, jnp.int32) for i in range(8)]
                for i in range(8):
                    for k in range(D // _LANES):
                        vals = plsc.load_gather(wbufs[0], [rows_k[k], cols[i]])
                        rbufs[0][t * 4 + (i >> 1), pl.ds((i & 1) * D + k * _LANES, _LANES)] = vals
            pltpu.sync_copy(
                rbufs[0].at[pl.ds(0, _CW // 2), :],
                out_hbm.at[pl.ds((start + BASE) * (_CW // 2), _CW // 2)],
            )

        @pl.when(wid == EXTRA)
        def _():
            t = tail_hbm.shape[0]
            pltpu.sync_copy(tail_hbm, rbufs[1].at[pl.ds(0, t), :])
            pltpu.sync_copy(
                rbufs[1].at[pl.ds(0, t), :],
                out_hbm.at[pl.ds(NF * (_CW // 2), t)],
            )

    return ka(tt, tail3)


@functools.partial(jax.jit, static_argnums=(3, 4))
def _emb_lookup(idx, pe4, table, seq_len, nch):
    D = table.shape[1]
    mesh = plsc.VectorSubcoreMesh(core_axis_name="c", subcore_axis_name="s")

    NBUF = 10
    H = 5

    @functools.partial(
        pl.kernel,
        mesh=mesh,
        compiler_params=pltpu.CompilerParams(use_tc_tiling_on_sc=False),
        out_type=jax.ShapeDtypeStruct((_NW * nch * _CW, D), jnp.float32),
        scratch_types=[
            pltpu.VMEM((nch, _CW), jnp.int32),
            pltpu.VMEM((4 * seq_len, D), jnp.float32),
            [pltpu.VMEM((_CW, D), jnp.float32) for _ in range(NBUF)],
            [pltpu.SemaphoreType.DMA for _ in range(NBUF)],
            [pltpu.SemaphoreType.DMA for _ in range(NBUF)],
        ],
    )
    def k(idx_hbm, pe4_hbm, table_hbm, out_hbm, idx_v, pe4_v, bufs, gsem, ssem):
        wid = lax.axis_index("s") * _NC + lax.axis_index("c")
        pltpu.sync_copy(idx_hbm.at[wid], idx_v)
        pltpu.sync_copy(pe4_hbm, pe4_v)

        def fire_gather(j):
            b = j % NBUF
            return pltpu.async_copy(table_hbm.at[idx_v.at[j]], bufs[b], gsem[b])

        def add_pe(j):
            buf = bufs[j % NBUF]
            phase = (j * _CW) % seq_len

            @plsc.parallel_loop(0, _CW, unroll=4)
            def row(r):
                for c in range(D // _LANES):
                    sl = pl.ds(c * _LANES, _LANES)
                    buf[r, sl] = buf[r, sl] + pe4_v[phase + r, sl]

        gh = [None] * nch
        sh = [None] * nch
        for j in range(H):
            gh[j] = fire_gather(j)
        for j in range(nch):
            b = j % NBUF
            gh[j].wait()
            add_pe(j)
            sh[j] = pltpu.async_copy(
                bufs[b], out_hbm.at[pl.ds((wid * nch + j) * _CW, _CW)], ssem[b]
            )
            jn = j + H
            if jn < nch:
                if jn - NBUF >= 0:
                    sh[jn - NBUF].wait()
                gh[jn] = fire_gather(jn)
        for j in range(max(0, nch - NBUF), nch):
            sh[j].wait()

    return k(idx, pe4, table)


def kernel(sequences, token_table):
    B, L = sequences.shape
    V, D = token_table.shape
    R = B * L
    assert R % (_NW * _CW) == 0
    nch = R // (_NW * _CW)

    VF = (V // _CW) * _CW
    tail3 = token_table[VF:].reshape((V - VF) * D // 128, 128)
    tab_lin = _relayout(token_table.T, tail3)
    tab_lin = tab_lin.reshape(V, D)

    idx = sequences.reshape(_NW, nch, _CW).astype(jnp.int32)
    pe = _pe_table(L, D)
    pe4 = jnp.concatenate([pe, pe, pe, pe], axis=0)
    out = _emb_lookup(idx, pe4, tab_lin, L, nch)
    return out.reshape(B, L, D)


# speedup vs baseline: 2.3781x; 1.3953x over previous
import functools

import jax
import jax.numpy as jnp
from jax import lax
from jax.experimental import pallas as pl
from jax.experimental.pallas import tpu as pltpu
from jax.experimental.pallas import tpu_sc as plsc

_NC = 2
_NS = 16
_NW = _NC * _NS
_LANES = 16
_CW = 128


def _pe_table(seq_len, dim):
    pos = jnp.arange(seq_len, dtype=jnp.float32)[:, None]
    i = jnp.arange(0, dim, 2, dtype=jnp.float32)
    div = jnp.exp(-(jnp.log(10000.0)) * i / dim)
    pe = jnp.zeros((seq_len, dim), dtype=jnp.float32)
    pe = pe.at[:, 0::2].set(jnp.sin(pos * div))
    pe = pe.at[:, 1::2].set(jnp.cos(pos * div))
    return pe


@functools.partial(jax.jit, static_argnums=(3, 4))
def _emb_lookup(idx, pe4, table, seq_len, nch):
    D = table.shape[1] // 2
    mesh = plsc.VectorSubcoreMesh(core_axis_name="c", subcore_axis_name="s")

    NBUF = 6
    H = 3

    @functools.partial(
        pl.kernel,
        mesh=mesh,
        compiler_params=pltpu.CompilerParams(use_tc_tiling_on_sc=False),
        out_type=jax.ShapeDtypeStruct((_NW * nch * _CW, D), jnp.float32),
        scratch_types=[
            pltpu.VMEM((nch, _CW), jnp.int32),
            pltpu.VMEM((4 * seq_len, D), jnp.float32),
            [pltpu.VMEM((_CW, 2 * D), jnp.float32) for _ in range(NBUF)],
            [pltpu.SemaphoreType.DMA for _ in range(NBUF)],
            [pltpu.SemaphoreType.DMA for _ in range(NBUF)],
        ],
    )
    def k(idx_hbm, pe4_hbm, table_hbm, out_hbm, idx_v, pe4_v, bufs, gsem, ssem):
        wid = lax.axis_index("s") * _NC + lax.axis_index("c")
        pltpu.sync_copy(idx_hbm.at[wid], idx_v)
        pltpu.sync_copy(pe4_hbm, pe4_v)

        def fire_gather(j):
            b = j % NBUF
            return pltpu.async_copy(table_hbm.at[idx_v.at[j]], bufs[b], gsem[b])

        def add_pe(j):
            buf = bufs[j % NBUF]
            phase = (j * _CW) % seq_len

            @plsc.parallel_loop(0, _CW, unroll=4)
            def row(r):
                for c in range(D // _LANES):
                    sl = pl.ds(c * _LANES, _LANES)
                    buf[r, sl] = buf[r, sl] + pe4_v[phase + r, sl]

        gh = [None] * nch
        sh = [None] * nch
        for j in range(H):
            gh[j] = fire_gather(j)
        for j in range(nch):
            b = j % NBUF
            gh[j].wait()
            add_pe(j)
            sh[j] = pltpu.async_copy(
                bufs[b].at[:, pl.ds(0, D)],
                out_hbm.at[pl.ds((wid * nch + j) * _CW, _CW)],
                ssem[b],
            )
            jn = j + H
            if jn < nch:
                if jn - NBUF >= 0:
                    sh[jn - NBUF].wait()
                gh[jn] = fire_gather(jn)
        for j in range(max(0, nch - NBUF), nch):
            sh[j].wait()

    return k(idx, pe4, table)


def kernel(sequences, token_table):
    B, L = sequences.shape
    V, D = token_table.shape
    R = B * L
    assert R % (_NW * _CW) == 0
    nch = R // (_NW * _CW)

    idx = sequences.reshape(_NW, nch, _CW).astype(jnp.int32)
    pe = _pe_table(L, D)
    pe4 = jnp.concatenate([pe, pe, pe, pe], axis=0)
    table_padded = jnp.pad(token_table, ((0, 0), (0, D)))
    out = _emb_lookup(idx, pe4, table_padded, L, nch)
    return out.reshape(B, L, D)
